# baseline (speedup 1.0000x reference)
import functools

import jax
import jax.numpy as jnp
from jax import lax
from jax.experimental import pallas as pl
from jax.experimental.pallas import tpu as pltpu
from jax.experimental.pallas import tpu_sc as plsc

_B = 4096
_V = 1024
_NC = 2
_NS = 16
_NW = _NC * _NS
_ROWS_PER_W = _B // _NW
_GRP = 16
_NGRP = _ROWS_PER_W // _GRP


def _hash_body(x_ref, idx_ref):
    x = x_ref[...]
    p = x[:, 0:8]
    for k in range(1, 16):
        p = p + x[:, 8 * k:8 * k + 8]
    q = p[:, 0:4] + p[:, 4:8]
    r = q[:, 0:2] + q[:, 2:4]
    m = (r[:, 0:1] + r[:, 1:2]) * (1.0 / 128.0)
    dec = (jnp.mod(m * 100.0, 1.0) * 10000.0).astype(jnp.int32)
    idx_ref[...] = (dec / 10000 * _V).astype(jnp.int32)


def _tanh_body(x_ref, o_ref):
    o_ref[...] = jnp.tanh(x_ref[...])


def _make_sc_gather():
    mesh = plsc.VectorSubcoreMesh(core_axis_name="c", subcore_axis_name="s")

    @functools.partial(
        pl.kernel,
        mesh=mesh,
        out_type=jax.ShapeDtypeStruct((_B, 3, 64, 64), jnp.float32),
        scratch_types=[
            pltpu.VMEM((_ROWS_PER_W,), jnp.int32),
            pltpu.SemaphoreType.DMA,
            pltpu.SemaphoreType.DMA,
        ],
    )
    def gather_kernel(table_hbm, idx_hbm, out_hbm, idx_v, isem, gsem):
        wid = lax.axis_index("s") * _NC + lax.axis_index("c")
        base = wid * _ROWS_PER_W
        pltpu.async_copy(idx_hbm.at[wid], idx_v, isem).wait()

        def issue_group(g):
            v = idx_v[pl.ds(g * _GRP, _GRP)]
            for t in range(_GRP):
                pltpu.async_copy(table_hbm.at[v[t]],
                                 out_hbm.at[base + g * _GRP + t], gsem)

        def drain_group(g):
            pltpu.make_async_copy(
                table_hbm.at[pl.ds(0, _GRP)],
                out_hbm.at[pl.ds(base, _GRP)], gsem).wait()

        issue_group(0)

        def body(g, carry):
            issue_group(g)
            drain_group(g - 1)
            return carry

        lax.fori_loop(1, _NGRP, body, 0)
        drain_group(_NGRP - 1)

    return gather_kernel


def kernel(input, images):
    assert input.shape == (_B, 128)
    assert images.shape == (_V, 3, 64, 64)

    idx = pl.pallas_call(
        _hash_body,
        out_shape=jax.ShapeDtypeStruct((_B, 1), jnp.int32),
    )(input)

    tanh_bank = pl.pallas_call(
        _tanh_body,
        grid=(16,),
        in_specs=[pl.BlockSpec((_V // 16, 3, 64, 64), lambda i: (i, 0, 0, 0))],
        out_specs=pl.BlockSpec((_V // 16, 3, 64, 64), lambda i: (i, 0, 0, 0)),
        out_shape=jax.ShapeDtypeStruct((_V, 3, 64, 64), jnp.float32),
    )(images)

    idx_r = idx.reshape(_NW, _ROWS_PER_W)
    return _make_sc_gather()(tanh_bank, idx_r)


# speedup vs baseline: 12.0326x; 12.0326x over previous
import functools

import jax
import jax.numpy as jnp
from jax import lax
from jax.experimental import pallas as pl
from jax.experimental.pallas import tpu as pltpu
from jax.experimental.pallas import tpu_sc as plsc

_B = 4096
_D = 3 * 64 * 64
_V = 1024
_NC = 2
_NS = 16
_NW = _NC * _NS
_ROWS_PER_W = _B // _NW
_CH = 4
_NCH = _ROWS_PER_W // _CH


def _hash_body(x_ref, idx_ref):
    x = x_ref[...]
    p = x[:, 0:8]
    for k in range(1, 16):
        p = p + x[:, 8 * k:8 * k + 8]
    q = p[:, 0:4] + p[:, 4:8]
    r = q[:, 0:2] + q[:, 2:4]
    m = (r[:, 0:1] + r[:, 1:2]) * (1.0 / 128.0)
    dec = (jnp.mod(m * 100.0, 1.0) * 10000.0).astype(jnp.int32)
    idx_ref[...] = (dec / 10000 * _V).astype(jnp.int32)


def _tanh_flatten_body(x_ref, o_ref):
    o_ref[...] = jnp.tanh(x_ref[...]).reshape(x_ref.shape[0], _D)


def _unflatten_body(x_ref, o_ref):
    o_ref[...] = x_ref[...].reshape(x_ref.shape[0], 3, 64, 64)


def _make_sc_gather():
    mesh = plsc.VectorSubcoreMesh(core_axis_name="c", subcore_axis_name="s")

    @functools.partial(
        pl.kernel,
        mesh=mesh,
        out_type=jax.ShapeDtypeStruct((_B, _D), jnp.float32),
        scratch_types=[
            pltpu.VMEM((_NCH, _CH), jnp.int32),
            pltpu.VMEM((_CH, _D), jnp.float32),
            pltpu.VMEM((_CH, _D), jnp.float32),
            pltpu.SemaphoreType.DMA,
            pltpu.SemaphoreType.DMA,
            pltpu.SemaphoreType.DMA,
            pltpu.SemaphoreType.DMA,
        ],
    )
    def gather_kernel(table_hbm, idx_hbm, out_hbm, idx_v,
                      buf0, buf1, gsem0, gsem1, wsem0, wsem1):
        wid = lax.axis_index("s") * _NC + lax.axis_index("c")
        base = wid * _ROWS_PER_W
        pltpu.sync_copy(idx_hbm.at[pl.ds(wid * _NCH, _NCH)], idx_v)

        def issue_g(j, buf, sem):
            pltpu.async_copy(table_hbm.at[idx_v.at[j]], buf, sem)

        def wait_g(j, buf, sem):
            pltpu.make_async_copy(table_hbm.at[idx_v.at[j]], buf, sem).wait()

        def issue_w(j, buf, sem):
            pltpu.async_copy(buf, out_hbm.at[pl.ds(base + j * _CH, _CH)], sem)

        def wait_w(j, buf, sem):
            pltpu.make_async_copy(
                buf, out_hbm.at[pl.ds(base + j * _CH, _CH)], sem).wait()

        issue_g(0, buf0, gsem0)
        issue_g(1, buf1, gsem1)

        def pair(i, carry):
            j = 2 * i
            wait_g(j, buf0, gsem0)
            issue_w(j, buf0, wsem0)
            wait_g(j + 1, buf1, gsem1)
            issue_w(j + 1, buf1, wsem1)
            wait_w(j, buf0, wsem0)
            issue_g(j + 2, buf0, gsem0)
            wait_w(j + 1, buf1, wsem1)
            issue_g(j + 3, buf1, gsem1)
            return carry

        lax.fori_loop(0, (_NCH - 2) // 2, pair, 0)

        j = _NCH - 2
        wait_g(j, buf0, gsem0)
        issue_w(j, buf0, wsem0)
        wait_g(j + 1, buf1, gsem1)
        issue_w(j + 1, buf1, wsem1)
        wait_w(j, buf0, wsem0)
        wait_w(j + 1, buf1, wsem1)

    return gather_kernel


def kernel(input, images):
    assert input.shape == (_B, 128)
    assert images.shape == (_V, 3, 64, 64)

    idx = pl.pallas_call(
        _hash_body,
        out_shape=jax.ShapeDtypeStruct((_B, 1), jnp.int32),
    )(input)

    tanh_bank = pl.pallas_call(
        _tanh_flatten_body,
        grid=(128,),
        in_specs=[pl.BlockSpec((8, 3, 64, 64), lambda i: (i, 0, 0, 0))],
        out_specs=pl.BlockSpec((8, _D), lambda i: (i, 0)),
        out_shape=jax.ShapeDtypeStruct((_V, _D), jnp.float32),
    )(images)

    idx2 = idx.reshape(_B // _CH, _CH)
    flat = _make_sc_gather()(tanh_bank, idx2)

    return pl.pallas_call(
        _unflatten_body,
        grid=(512,),
        in_specs=[pl.BlockSpec((8, _D), lambda i: (i, 0))],
        out_specs=pl.BlockSpec((8, 3, 64, 64), lambda i: (i, 0, 0, 0)),
        out_shape=jax.ShapeDtypeStruct((_B, 3, 64, 64), jnp.float32),
    )(flat)


# speedup vs baseline: 12.2996x; 1.0222x over previous
import functools

import jax
import jax.numpy as jnp
from jax import lax
from jax.experimental import pallas as pl
from jax.experimental.pallas import tpu as pltpu
from jax.experimental.pallas import tpu_sc as plsc

_B = 4096
_D = 3 * 64 * 64
_V = 1024
_NC = 2
_NS = 16
_NW = _NC * _NS
_ROWS_PER_W = _B // _NW
_CH = 8
_NCH = _ROWS_PER_W // _CH


def _hash_body(x_ref, idx_ref):
    x = x_ref[...]
    p = x[:, 0:8]
    for k in range(1, 16):
        p = p + x[:, 8 * k:8 * k + 8]
    q = p[:, 0:4] + p[:, 4:8]
    r = q[:, 0:2] + q[:, 2:4]
    m = (r[:, 0:1] + r[:, 1:2]) * (1.0 / 128.0)
    dec = (jnp.mod(m * 100.0, 1.0) * 10000.0).astype(jnp.int32)
    idx_ref[...] = (dec / 10000 * _V).astype(jnp.int32)


def _tanh_body(x_ref, o_ref):
    o_ref[...] = jnp.tanh(x_ref[...])


def _make_sc_gather():
    mesh = plsc.VectorSubcoreMesh(core_axis_name="c", subcore_axis_name="s")

    @functools.partial(
        pl.kernel,
        mesh=mesh,
        out_type=jax.ShapeDtypeStruct((_B, _D // 2), jnp.float32),
        scratch_types=[
            pltpu.VMEM((_NCH, _CH), jnp.int32),
            pltpu.VMEM((_CH, _D // 2), jnp.float32),
            pltpu.VMEM((_CH, _D // 2), jnp.float32),
            pltpu.SemaphoreType.DMA,
            pltpu.SemaphoreType.DMA,
            pltpu.SemaphoreType.DMA,
            pltpu.SemaphoreType.DMA,
        ],
    )
    def gather_kernel(table_hbm, idx_hbm, out_hbm, idx_v,
                      buf0, buf1, gsem0, gsem1, wsem0, wsem1):
        wid = lax.axis_index("s") * _NC + lax.axis_index("c")
        base = wid * _ROWS_PER_W
        pltpu.sync_copy(idx_hbm.at[pl.ds(wid * _NCH, _NCH)], idx_v)

        def issue_g(j, buf, sem):
            pltpu.async_copy(table_hbm.at[idx_v.at[j]], buf, sem)

        def wait_g(j, buf, sem):
            pltpu.make_async_copy(table_hbm.at[idx_v.at[j]], buf, sem).wait()

        def issue_w(j, buf, sem):
            pltpu.async_copy(buf, out_hbm.at[pl.ds(base + j * _CH, _CH)], sem)

        def wait_w(j, buf, sem):
            pltpu.make_async_copy(
                buf, out_hbm.at[pl.ds(base + j * _CH, _CH)], sem).wait()

        issue_g(0, buf0, gsem0)
        issue_g(1, buf1, gsem1)

        def pair(i, carry):
            j = 2 * i
            wait_g(j, buf0, gsem0)
            issue_w(j, buf0, wsem0)
            wait_g(j + 1, buf1, gsem1)
            issue_w(j + 1, buf1, wsem1)
            wait_w(j, buf0, wsem0)
            issue_g(j + 2, buf0, gsem0)
            wait_w(j + 1, buf1, wsem1)
            issue_g(j + 3, buf1, gsem1)
            return carry

        lax.fori_loop(0, (_NCH - 2) // 2, pair, 0)

        j = _NCH - 2
        wait_g(j, buf0, gsem0)
        issue_w(j, buf0, wsem0)
        wait_g(j + 1, buf1, gsem1)
        issue_w(j + 1, buf1, wsem1)
        wait_w(j, buf0, wsem0)
        wait_w(j + 1, buf1, wsem1)

    return gather_kernel


def kernel(input, images):
    assert input.shape == (_B, 128)
    assert images.shape == (_V, 3, 64, 64)

    idx = pl.pallas_call(
        _hash_body,
        out_shape=jax.ShapeDtypeStruct((_B, 1), jnp.int32),
    )(input)

    bank16 = images.astype(jnp.bfloat16).reshape(_V, _D)
    tanh_bank = pl.pallas_call(
        _tanh_body,
        grid=(16,),
        in_specs=[pl.BlockSpec((_V // 16, _D), lambda i: (i, 0))],
        out_specs=pl.BlockSpec((_V // 16, _D), lambda i: (i, 0)),
        out_shape=jax.ShapeDtypeStruct((_V, _D), jnp.bfloat16),
    )(bank16)

    packed = lax.bitcast_convert_type(
        tanh_bank.reshape(_V, _D // 2, 2), jnp.float32)

    idx2 = idx.reshape(_B // _CH, _CH)
    flat = _make_sc_gather()(packed, idx2)

    out16 = lax.bitcast_convert_type(flat, jnp.bfloat16).reshape(_B, _D)
    return out16.reshape(_B, 3, 64, 64).astype(jnp.float32)


# speedup vs baseline: 29.1732x; 2.3719x over previous
import functools

import jax
import jax.numpy as jnp
from jax import lax
from jax.experimental import pallas as pl
from jax.experimental.pallas import tpu as pltpu
from jax.experimental.pallas import tpu_sc as plsc

_B = 4096
_D = 3 * 64 * 64
_V = 1024
_NC = 2
_NS = 16
_NW = _NC * _NS
_ROWS_PER_W = _B // _NW
_CH = 4
_NCH = _ROWS_PER_W // _CH


def _hash_body(x_ref, idx_ref):
    x = x_ref[...]
    p = x[:, 0:8]
    for k in range(1, 16):
        p = p + x[:, 8 * k:8 * k + 8]
    q = p[:, 0:4] + p[:, 4:8]
    r = q[:, 0:2] + q[:, 2:4]
    m = (r[:, 0:1] + r[:, 1:2]) * (1.0 / 128.0)
    dec = (jnp.mod(m * 100.0, 1.0) * 10000.0).astype(jnp.int32)
    idx_ref[...] = (dec / 10000 * _V).astype(jnp.int32)


def _tanh_body(x_ref, o_ref):
    o_ref[...] = jnp.tanh(x_ref[...])


def _make_sc_gather():
    mesh = plsc.VectorSubcoreMesh(core_axis_name="c", subcore_axis_name="s")

    @functools.partial(
        pl.kernel,
        mesh=mesh,
        out_type=jax.ShapeDtypeStruct((_B, _D), jnp.float32),
        scratch_types=[
            pltpu.VMEM((_NCH, _CH), jnp.int32),
            pltpu.VMEM((_CH, _D), jnp.float32),
            pltpu.VMEM((_CH, _D), jnp.float32),
            pltpu.SemaphoreType.DMA,
            pltpu.SemaphoreType.DMA,
            pltpu.SemaphoreType.DMA,
            pltpu.SemaphoreType.DMA,
        ],
    )
    def gather_kernel(table_hbm, idx_hbm, out_hbm, idx_v,
                      buf0, buf1, gsem0, gsem1, wsem0, wsem1):
        wid = lax.axis_index("s") * _NC + lax.axis_index("c")
        base = wid * _ROWS_PER_W
        pltpu.sync_copy(idx_hbm.at[pl.ds(wid * _NCH, _NCH)], idx_v)

        def issue_g(j, buf, sem):
            pltpu.async_copy(table_hbm.at[idx_v.at[j]], buf, sem)

        def wait_g(j, buf, sem):
            pltpu.make_async_copy(table_hbm.at[idx_v.at[j]], buf, sem).wait()

        def issue_w(j, buf, sem):
            pltpu.async_copy(buf, out_hbm.at[pl.ds(base + j * _CH, _CH)], sem)

        def wait_w(j, buf, sem):
            pltpu.make_async_copy(
                buf, out_hbm.at[pl.ds(base + j * _CH, _CH)], sem).wait()

        issue_g(0, buf0, gsem0)
        issue_g(1, buf1, gsem1)

        def pair(i, carry):
            j = 2 * i
            wait_g(j, buf0, gsem0)
            issue_w(j, buf0, wsem0)
            wait_g(j + 1, buf1, gsem1)
            issue_w(j + 1, buf1, wsem1)
            wait_w(j, buf0, wsem0)
            issue_g(j + 2, buf0, gsem0)
            wait_w(j + 1, buf1, wsem1)
            issue_g(j + 3, buf1, gsem1)
            return carry

        lax.fori_loop(0, (_NCH - 2) // 2, pair, 0)

        j = _NCH - 2
        wait_g(j, buf0, gsem0)
        issue_w(j, buf0, wsem0)
        wait_g(j + 1, buf1, gsem1)
        issue_w(j + 1, buf1, wsem1)
        wait_w(j, buf0, wsem0)
        wait_w(j + 1, buf1, wsem1)

    return gather_kernel


def kernel(input, images):
    assert input.shape == (_B, 128)
    assert images.shape == (_V, 3, 64, 64)

    idx = pl.pallas_call(
        _hash_body,
        out_shape=jax.ShapeDtypeStruct((_B, 1), jnp.int32),
    )(input)

    bank = images.reshape(_V, _D)
    tanh_bank = pl.pallas_call(
        _tanh_body,
        grid=(16,),
        in_specs=[pl.BlockSpec((_V // 16, _D), lambda i: (i, 0))],
        out_specs=pl.BlockSpec((_V // 16, _D), lambda i: (i, 0)),
        out_shape=jax.ShapeDtypeStruct((_V, _D), jnp.float32),
    )(bank)

    idx2 = idx.reshape(_B // _CH, _CH)
    flat = _make_sc_gather()(tanh_bank, idx2)
    return flat.reshape(_B, 3, 64, 64)
